# baseline (speedup 1.0000x reference)
import functools

import jax
import jax.numpy as jnp
from jax import lax
from jax.experimental import pallas as pl
from jax.experimental.pallas import tpu as pltpu
from jax.experimental.pallas import tpu_sc as plsc

L = 16

_INT_MAX = 2**31 - 1


def _worker_id():
    return lax.axis_index("s") * lax.axis_size("c") + lax.axis_index("c")


def _better(val, idx, best_val, best_idx):
    upd = (val < best_val) | ((val == best_val) & (idx < best_idx))
    return jnp.where(upd, val, best_val), jnp.where(upd, idx, best_idx)


def _make_stage1(K, NW, CH_G, NC):
    GT = K // L
    BASE_G = GT // NW
    EXTRA = GT % NW
    NCHUNK = -(-(BASE_G + (1 if EXTRA else 0)) // CH_G)
    CH_ROWS = CH_G * L

    mesh = plsc.VectorSubcoreMesh(core_axis_name="c", subcore_axis_name="s")

    @functools.partial(
        pl.kernel,
        out_type=(
            jax.ShapeDtypeStruct((NW, L), jnp.float32),
            jax.ShapeDtypeStruct((NW, L), jnp.int32),
        ),
        mesh=mesh,
        compiler_params=pltpu.CompilerParams(
            needs_layout_passes=False, use_tc_tiling_on_sc=False),
        scratch_types=[
            pltpu.VMEM((CH_ROWS, L), jnp.float32),
            pltpu.VMEM((CH_ROWS, L), jnp.float32),
            pltpu.VMEM((L,), jnp.float32),
            pltpu.VMEM((L,), jnp.float32),
            pltpu.VMEM((L,), jnp.int32),
            pltpu.SemaphoreType.DMA,
            pltpu.SemaphoreType.DMA,
        ],
    )
    def stage1(q_hbm, obs_hbm, oval_hbm, oidx_hbm,
               buf0, buf1, qv, sval, sidx, sem0, sem1):
        w = _worker_id()
        g0 = w * BASE_G + jnp.minimum(w, EXTRA)
        ng = BASE_G + jnp.where(w < EXTRA, 1, 0)
        row0 = g0 * L
        row_hi = (g0 + ng) * L - CH_ROWS

        pltpu.sync_copy(q_hbm.at[0], qv)

        iota = lax.iota(jnp.int32, L)
        colv = [(iota + j) & (L - 1) for j in range(L)]
        qrot = [plsc.load_gather(qv, [colv[j]]) for j in range(L)]

        bufs = (buf0, buf1)
        sems = (sem0, sem1)

        def chunk_base(c):
            return jnp.minimum(row0 + c * CH_ROWS, row_hi)

        def start(c):
            return pltpu.async_copy(
                obs_hbm.at[pl.ds(chunk_base(c), CH_ROWS)], bufs[c % 2],
                sems[c % 2])

        best_val = jnp.full((L,), jnp.inf, jnp.float32)
        best_idx = jnp.zeros((L,), jnp.int32)

        cp = start(0)
        for c in range(NCHUNK):
            if c + 1 < NCHUNK:
                nxt = start(c + 1)
            cp.wait()
            buf = bufs[c % 2]
            base_row = chunk_base(c)

            def group(g, carry):
                bv, bi = carry
                rowvec = g * L + iota
                parts = []
                for j in range(L):
                    x = plsc.load_gather(buf, [rowvec, colv[j]])
                    t = x - qrot[j]
                    parts.append(t * t)
                while len(parts) > 1:
                    parts = [parts[i] + parts[i + 1]
                             for i in range(0, len(parts), 2)]
                dist = parts[0]
                gidx = base_row + rowvec
                return _better(dist, gidx, bv, bi)

            best_val, best_idx = lax.fori_loop(
                0, CH_G, group, (best_val, best_idx))
            if c + 1 < NCHUNK:
                cp = nxt

        sval[...] = best_val
        sidx[...] = best_idx
        pltpu.sync_copy(sval, oval_hbm.at[w])
        pltpu.sync_copy(sidx, oidx_hbm.at[w])

    return stage1


def _make_stage2(K, NW):
    mesh = plsc.VectorSubcoreMesh(core_axis_name="c", subcore_axis_name="s")

    @functools.partial(
        pl.kernel,
        out_type=(
            jax.ShapeDtypeStruct((1, L), jnp.float32),
            jax.ShapeDtypeStruct((1, L), jnp.float32),
        ),
        mesh=mesh,
        compiler_params=pltpu.CompilerParams(
            needs_layout_passes=False, use_tc_tiling_on_sc=False),
        scratch_types=[
            pltpu.VMEM((NW, L), jnp.float32),
            pltpu.VMEM((NW, L), jnp.int32),
            pltpu.VMEM((L, L), jnp.float32),
            pltpu.VMEM((L, L), jnp.float32),
            pltpu.SemaphoreType.DMA,
        ],
    )
    def stage2(val_hbm, idx_hbm, pv_hbm, pm_hbm, ovel_hbm, omask_hbm,
               vals_v, idxs_v, rowv, rowm, sem):
        w = _worker_id()

        @pl.when(w == 0)
        def _():
            pltpu.sync_copy(val_hbm, vals_v)
            pltpu.sync_copy(idx_hbm, idxs_v)
            best_val = vals_v[0]
            best_idx = idxs_v[0]
            for i in range(1, NW):
                best_val, best_idx = _better(
                    vals_v[i], idxs_v[i], best_val, best_idx)
            m = jnp.min(best_val)
            cand = jnp.where(best_val == m, best_idx, _INT_MAX)
            ind = jnp.min(cand)
            ivec = jnp.broadcast_to(ind, (L,))
            pltpu.async_copy(pv_hbm.at[ivec], rowv, sem).wait()
            pltpu.async_copy(pm_hbm.at[ivec], rowm, sem).wait()
            pltpu.sync_copy(rowv.at[0], ovel_hbm.at[0])
            pltpu.sync_copy(rowm.at[0], omask_hbm.at[0])

    return stage2


def kernel(in_vel, obs_vel, pred_vel, pred_mask):
    K, D = obs_vel.shape
    assert D == L and K % L == 0
    info = plsc.get_sparse_core_info()
    NC, NS = info.num_cores, info.num_subcores
    NW = NC * NS
    CH_G = 123

    stage1 = _make_stage1(K, NW, CH_G, NC)
    vals, idxs = stage1(in_vel, obs_vel)
    stage2 = _make_stage2(K, NW)
    best_vel, best_mask = stage2(vals, idxs, pred_vel, pred_mask)
    return best_vel, best_mask


# speedup vs baseline: 1.0001x; 1.0001x over previous
import functools

import jax
import jax.numpy as jnp
from jax import lax
from jax.experimental import pallas as pl
from jax.experimental.pallas import tpu as pltpu
from jax.experimental.pallas import tpu_sc as plsc

L = 16

_INT_MAX = 2**31 - 1

_SC_PARAMS = dict(
    needs_layout_passes=False,
    use_tc_tiling_on_sc=False,
)


def _worker_id():
    return lax.axis_index("s") * lax.axis_size("c") + lax.axis_index("c")


def _better(val, idx, best_val, best_idx):
    upd = (val < best_val) | ((val == best_val) & (idx < best_idx))
    return jnp.where(upd, val, best_val), jnp.where(upd, idx, best_idx)


def _make_stage1(K, NW, CH_G):
    GT = K // L
    BASE_G = GT // NW
    EXTRA = GT % NW
    NCHUNK = -(-(BASE_G + (1 if EXTRA else 0)) // CH_G)
    CH_W = CH_G * L * L

    mesh = plsc.VectorSubcoreMesh(core_axis_name="c", subcore_axis_name="s")

    @functools.partial(
        pl.kernel,
        out_type=(
            jax.ShapeDtypeStruct((NW * L,), jnp.float32),
            jax.ShapeDtypeStruct((NW * L,), jnp.int32),
        ),
        mesh=mesh,
        compiler_params=pltpu.CompilerParams(**_SC_PARAMS),
        scratch_types=[
            pltpu.VMEM((CH_W,), jnp.float32),
            pltpu.VMEM((CH_W,), jnp.float32),
            pltpu.VMEM((L,), jnp.float32),
            pltpu.VMEM((L,), jnp.float32),
            pltpu.VMEM((L,), jnp.int32),
            pltpu.SemaphoreType.DMA,
            pltpu.SemaphoreType.DMA,
        ],
    )
    def stage1(q_hbm, obs_hbm, oval_hbm, oidx_hbm,
               buf0, buf1, qv, sval, sidx, sem0, sem1):
        w = _worker_id()
        g0 = w * BASE_G + jnp.minimum(w, EXTRA)
        ng = BASE_G + jnp.where(w < EXTRA, 1, 0)
        word0 = g0 * (L * L)
        word_hi = (g0 + ng) * (L * L) - CH_W

        pltpu.sync_copy(q_hbm, qv)

        iota = lax.iota(jnp.int32, L)
        colv = [(iota + j) & (L - 1) for j in range(L)]
        offs = [iota * L + colv[j] for j in range(L)]
        qrot = [plsc.load_gather(qv, [colv[j]]) for j in range(L)]

        bufs = (buf0, buf1)
        sems = (sem0, sem1)

        def chunk_base(c):
            return jnp.minimum(word0 + c * CH_W, word_hi)

        def start(c):
            return pltpu.async_copy(
                obs_hbm.at[pl.ds(chunk_base(c), CH_W)], bufs[c % 2],
                sems[c % 2])

        best_val = jnp.full((L,), jnp.inf, jnp.float32)
        best_idx = jnp.zeros((L,), jnp.int32)

        cp = start(0)
        for c in range(NCHUNK):
            if c + 1 < NCHUNK:
                nxt = start(c + 1)
            cp.wait()
            buf = bufs[c % 2]
            row_base = chunk_base(c) // L

            def group(g, carry):
                bv, bi = carry
                base = g * (L * L)
                parts = []
                for j in range(L):
                    x = plsc.load_gather(buf, [base + offs[j]])
                    t = x - qrot[j]
                    parts.append(t * t)
                while len(parts) > 1:
                    parts = [parts[i] + parts[i + 1]
                             for i in range(0, len(parts), 2)]
                dist = parts[0]
                gidx = row_base + g * L + iota
                return _better(dist, gidx, bv, bi)

            best_val, best_idx = lax.fori_loop(
                0, CH_G, group, (best_val, best_idx))
            if c + 1 < NCHUNK:
                cp = nxt

        sval[...] = best_val
        sidx[...] = best_idx
        pltpu.sync_copy(sval, oval_hbm.at[pl.ds(w * L, L)])
        pltpu.sync_copy(sidx, oidx_hbm.at[pl.ds(w * L, L)])

    return stage1


def _make_stage2(NW):
    mesh = plsc.VectorSubcoreMesh(core_axis_name="c", subcore_axis_name="s")

    @functools.partial(
        pl.kernel,
        out_type=(
            jax.ShapeDtypeStruct((L,), jnp.float32),
            jax.ShapeDtypeStruct((L,), jnp.float32),
        ),
        mesh=mesh,
        compiler_params=pltpu.CompilerParams(**_SC_PARAMS),
        scratch_types=[
            pltpu.VMEM((NW * L,), jnp.float32),
            pltpu.VMEM((NW * L,), jnp.int32),
            pltpu.VMEM((L,), jnp.float32),
            pltpu.VMEM((L,), jnp.float32),
            pltpu.SemaphoreType.DMA,
        ],
    )
    def stage2(val_hbm, idx_hbm, pv_hbm, pm_hbm, ovel_hbm, omask_hbm,
               vals_v, idxs_v, rowv, rowm, sem):
        w = _worker_id()

        @pl.when(w == 0)
        def _():
            pltpu.sync_copy(val_hbm, vals_v)
            pltpu.sync_copy(idx_hbm, idxs_v)
            best_val = vals_v[pl.ds(0, L)]
            best_idx = idxs_v[pl.ds(0, L)]
            for i in range(1, NW):
                best_val, best_idx = _better(
                    vals_v[pl.ds(i * L, L)], idxs_v[pl.ds(i * L, L)],
                    best_val, best_idx)
            m = jnp.min(best_val)
            cand = jnp.where(best_val == m, best_idx, _INT_MAX)
            ind = jnp.min(cand)
            ivec = ind * L + lax.iota(jnp.int32, L)
            pltpu.async_copy(pv_hbm.at[ivec], rowv, sem).wait()
            pltpu.async_copy(pm_hbm.at[ivec], rowm, sem).wait()
            pltpu.sync_copy(rowv, ovel_hbm)
            pltpu.sync_copy(rowm, omask_hbm)

    return stage2


def kernel(in_vel, obs_vel, pred_vel, pred_mask):
    K, D = obs_vel.shape
    assert D == L and K % L == 0
    info = plsc.get_sparse_core_info()
    NW = info.num_cores * info.num_subcores
    CH_G = 123

    q = in_vel.reshape(L)
    obs = obs_vel.reshape(-1)
    pv = pred_vel.reshape(-1)
    pm = pred_mask.reshape(-1)

    vals, idxs = _make_stage1(K, NW, CH_G)(q, obs)
    best_vel, best_mask = _make_stage2(NW)(vals, idxs, pv, pm)
    return best_vel.reshape(1, L), best_mask.reshape(1, L)


# speedup vs baseline: 18.6269x; 18.6257x over previous
import functools

import jax
import jax.numpy as jnp
from jax import lax
from jax.experimental import pallas as pl
from jax.experimental.pallas import tpu as pltpu
from jax.experimental.pallas import tpu_sc as plsc

L = 16
BLK = 128

_INT_MAX = 2**31 - 1

_SC_PARAMS = dict(
    needs_layout_passes=False,
    use_tc_tiling_on_sc=True,
    disable_bounds_checks=True,
)


def _worker_id():
    return lax.axis_index("s") * lax.axis_size("c") + lax.axis_index("c")


def _take16(v, idx):
    return lax.gather(
        v, idx[:, None],
        dimension_numbers=lax.GatherDimensionNumbers(
            offset_dims=(), collapsed_slice_dims=(0,), start_index_map=(0,)),
        slice_sizes=(1,),
        mode=lax.GatherScatterMode.PROMISE_IN_BOUNDS)


def _better(val, idx, best_val, best_idx):
    upd = (val < best_val) | ((val == best_val) & (idx < best_idx))
    return jnp.where(upd, val, best_val), jnp.where(upd, idx, best_idx)


def _make_stage1(K, NW, CH_B):
    NB = -(-K // BLK)
    BASE_B = NB // NW
    EXTRA = NB % NW
    NCHUNK = -(-(BASE_B + (1 if EXTRA else 0)) // CH_B)
    CH_C = CH_B * BLK

    mesh = plsc.VectorSubcoreMesh(core_axis_name="c", subcore_axis_name="s")

    @functools.partial(
        pl.kernel,
        out_type=(
            jax.ShapeDtypeStruct((NW, L), jnp.float32),
            jax.ShapeDtypeStruct((NW, L), jnp.int32),
        ),
        mesh=mesh,
        compiler_params=pltpu.CompilerParams(**_SC_PARAMS),
        scratch_types=[
            pltpu.VMEM((L, CH_C), jnp.float32),
            pltpu.VMEM((L, CH_C), jnp.float32),
            pltpu.VMEM((1, L), jnp.float32),
            pltpu.VMEM((L,), jnp.float32),
            pltpu.VMEM((L,), jnp.int32),
            pltpu.SemaphoreType.DMA,
            pltpu.SemaphoreType.DMA,
        ],
    )
    def stage1(q_hbm, obs_hbm, oval_hbm, oidx_hbm,
               buf0, buf1, qv, sval, sidx, sem0, sem1):
        w = _worker_id()
        b0 = w * BASE_B + jnp.minimum(w, EXTRA)
        nb = BASE_B + jnp.where(w < EXTRA, 1, 0)
        col0 = b0 * BLK
        col_hi = (b0 + nb) * BLK - CH_C

        pltpu.sync_copy(q_hbm, qv)
        qvec = qv[0]
        qs = [jnp.full((L,), qvec[f], jnp.float32) for f in range(L)]

        iota = lax.iota(jnp.int32, L)

        bufs = (buf0, buf1)
        sems = (sem0, sem1)

        def chunk_base(c):
            return jnp.minimum(col0 + c * CH_C, col_hi)

        def start(c):
            return pltpu.async_copy(
                obs_hbm.at[:, pl.ds(chunk_base(c), CH_C)], bufs[c % 2],
                sems[c % 2])

        best_val = jnp.full((L,), jnp.inf, jnp.float32)
        best_idx = jnp.zeros((L,), jnp.int32)

        cp = start(0)
        for c in range(NCHUNK):
            if c + 1 < NCHUNK:
                nxt = start(c + 1)
            cp.wait()
            buf = bufs[c % 2]
            base_idx = chunk_base(c) + iota

            def group(g, carry):
                bv, bi = carry
                r = g * L
                parts = []
                for f in range(L):
                    t = buf[f, pl.ds(r, L)] - qs[f]
                    parts.append(t * t)
                while len(parts) > 1:
                    parts = [parts[i] + parts[i + 1]
                             for i in range(0, len(parts), 2)]
                gidx = base_idx + r
                dist = jnp.where(gidx < K, parts[0], jnp.inf)
                return _better(dist, gidx, bv, bi)

            best_val, best_idx = lax.fori_loop(
                0, CH_C // L, group, (best_val, best_idx))
            if c + 1 < NCHUNK:
                cp = nxt

        sval[...] = best_val
        sidx[...] = best_idx
        pltpu.sync_copy(sval, oval_hbm.at[w])
        pltpu.sync_copy(sidx, oidx_hbm.at[w])

    return stage1


def _make_stage2(K, NW):
    mesh = plsc.VectorSubcoreMesh(core_axis_name="c", subcore_axis_name="s")

    @functools.partial(
        pl.kernel,
        out_type=(
            jax.ShapeDtypeStruct((1, L), jnp.float32),
            jax.ShapeDtypeStruct((1, L), jnp.float32),
        ),
        mesh=mesh,
        compiler_params=pltpu.CompilerParams(**_SC_PARAMS),
        scratch_types=[
            pltpu.VMEM((NW, L), jnp.float32),
            pltpu.VMEM((NW, L), jnp.int32),
            pltpu.VMEM((L, BLK), jnp.float32),
            pltpu.VMEM((L, BLK), jnp.float32),
            pltpu.VMEM((L,), jnp.float32),
            pltpu.VMEM((L,), jnp.float32),
            pltpu.SemaphoreType.DMA,
        ],
    )
    def stage2(val_hbm, idx_hbm, pv_hbm, pm_hbm, ovel_hbm, omask_hbm,
               vals_v, idxs_v, blkv, blkm, rowv, rowm, sem):
        w = _worker_id()

        @pl.when(w == 0)
        def _():
            pltpu.sync_copy(val_hbm, vals_v)
            pltpu.sync_copy(idx_hbm, idxs_v)
            best_val = vals_v[0]
            best_idx = idxs_v[0]
            for i in range(1, NW):
                best_val, best_idx = _better(
                    vals_v[i], idxs_v[i], best_val, best_idx)
            m = jnp.min(best_val)
            cand = jnp.where(best_val == m, best_idx, _INT_MAX)
            ind = jnp.min(cand)
            blk0 = (ind // BLK) * BLK
            off = ind - blk0
            pltpu.async_copy(pv_hbm.at[:, pl.ds(blk0, BLK)], blkv, sem).wait()
            pltpu.async_copy(pm_hbm.at[:, pl.ds(blk0, BLK)], blkm, sem).wait()
            iota = lax.iota(jnp.int32, L)
            off_al = (off // L) * L
            omv = jnp.broadcast_to(off - off_al, (L,))
            out_v = jnp.zeros((L,), jnp.float32)
            out_m = jnp.zeros((L,), jnp.float32)
            for f in range(L):
                tv = _take16(blkv[f, pl.ds(off_al, L)], omv)
                tm = _take16(blkm[f, pl.ds(off_al, L)], omv)
                out_v = jnp.where(iota == f, tv, out_v)
                out_m = jnp.where(iota == f, tm, out_m)
            rowv[...] = out_v
            rowm[...] = out_m
            pltpu.sync_copy(rowv, ovel_hbm.at[0])
            pltpu.sync_copy(rowm, omask_hbm.at[0])

    return stage2


def kernel(in_vel, obs_vel, pred_vel, pred_mask):
    K, D = obs_vel.shape
    assert D == L
    info = plsc.get_sparse_core_info()
    NW = info.num_cores * info.num_subcores
    CH_B = 16

    obs_t = obs_vel.T
    pv_t = pred_vel.T
    pm_t = pred_mask.T

    vals, idxs = _make_stage1(K, NW, CH_B)(in_vel, obs_t)
    best_vel, best_mask = _make_stage2(K, NW)(vals, idxs, pv_t, pm_t)
    return best_vel, best_mask
